# Initial kernel scaffold; baseline (speedup 1.0000x reference)
import jax
import jax.numpy as jnp
from jax.experimental import pallas as pl


def kernel(input_or_shape, pos_table):
    raise NotImplementedError("write your pallas kernel here")


# speedup vs baseline: 5.0478x; 5.0478x over previous
import jax
import jax.numpy as jnp
from jax.experimental import pallas as pl


def kernel(input_or_shape, pos_table):
    batch, seq_len = input_or_shape.shape
    max_pos, hidden = pos_table.shape

    block_s = 512

    def body(tab_ref, out_ref):
        out_ref[...] = jnp.broadcast_to(tab_ref[...][None], (batch,) + tab_ref.shape)

    return pl.pallas_call(
        body,
        grid=(seq_len // block_s,),
        in_specs=[pl.BlockSpec((block_s, hidden), lambda i: (i, 0))],
        out_specs=pl.BlockSpec((batch, block_s, hidden), lambda i: (0, i, 0)),
        out_shape=jax.ShapeDtypeStruct((batch, seq_len, hidden), pos_table.dtype),
    )(pos_table)
